# baseline (speedup 1.0000x reference)
import functools

import jax
import jax.numpy as jnp
from jax import lax
from jax.experimental import pallas as pl
from jax.experimental.pallas import tpu as pltpu
from jax.experimental.pallas import tpu_sc as plsc

N_NODES = 10000
N_EDGES = 320000
IN_FEATS = 128
HIDDEN = 128
NUM_CLASSES = 64

NC = 2
NS = 16
NW = NC * NS
CHUNK = 128
BLK = 4
NBLK = 20
CHUNKS = BLK * NBLK
PAIRS = CHUNKS // 2
E_PAD = NW * CHUNKS * CHUNK
ROWS_PER_TILE = 640
N_PAD = NS * ROWS_PER_TILE
INIT_STEPS = ROWS_PER_TILE // CHUNK


def _sc_aggregate(d, with_cnt):
    mesh = plsc.VectorSubcoreMesh(core_axis_name="c", subcore_axis_name="s")
    out_type = [jax.ShapeDtypeStruct((NC, N_PAD, d), jnp.float32)]
    scratch = [
        pltpu.VMEM_SHARED((N_PAD, d), jnp.float32),
        pltpu.VMEM((2, BLK, 2, CHUNK), jnp.int32),
        pltpu.VMEM((2, CHUNK, d), jnp.float32),
        pltpu.SemaphoreType.DMA,
        pltpu.SemaphoreType.DMA,
        pltpu.SemaphoreType.DMA,
        pltpu.SemaphoreType.DMA,
    ]
    if with_cnt:
        out_type.append(jax.ShapeDtypeStruct((NC, N_PAD, 16), jnp.float32))
        scratch += [
            pltpu.VMEM_SHARED((N_PAD, 16), jnp.float32),
            pltpu.VMEM((CHUNK, 16), jnp.float32),
            pltpu.SemaphoreType.DMA,
            pltpu.SemaphoreType.DMA,
        ]

    def body(y, eR, z, z16, ones16, out, cntout, acc, idxb, rows,
             sg0, sg1, ss0, ss1, cntacc=None, ones_v=None, sc0=None,
             sc1=None):
        c = lax.axis_index("c")
        s = lax.axis_index("s")
        w = c * NS + s
        off = s * ROWS_PER_TILE

        pltpu.sync_copy(z, rows.at[0])
        for t in range(INIT_STEPS):
            pltpu.sync_copy(rows.at[0], acc.at[pl.ds(off + t * CHUNK, CHUNK)])
        if with_cnt:
            pltpu.sync_copy(z16, ones_v)
            for t in range(INIT_STEPS):
                pltpu.sync_copy(ones_v, cntacc.at[pl.ds(off + t * CHUNK, CHUNK)])
            pltpu.sync_copy(ones16, ones_v)
        plsc.subcore_barrier()

        pltpu.sync_copy(eR.at[w, 0], idxb.at[0])
        pltpu.async_copy(y.at[idxb.at[0, 0, 0]], rows.at[0], sg0)
        pltpu.async_copy(y.at[idxb.at[0, 1, 0]], rows.at[1], sg1)

        def pair(j2, carry):
            blk = j2 // 2
            @pl.when(jnp.logical_and(lax.rem(j2, 2) == 0, blk + 1 < NBLK))
            def _():
                pltpu.sync_copy(eR.at[w, blk + 1],
                                idxb.at[lax.rem(blk + 1, 2)])
            bs = lax.rem(blk, 2)
            ca = 2 * lax.rem(j2, 2)
            ia_s = idxb.at[bs, ca, 0]
            ia_d = idxb.at[bs, ca, 1]
            ib_s = idxb.at[bs, ca + 1, 0]
            ib_d = idxb.at[bs, ca + 1, 1]
            pltpu.make_async_copy(y.at[ia_s], rows.at[0], sg0).wait()
            pltpu.async_copy(rows.at[0], acc.at[ia_d], ss0, add=True)
            if with_cnt:
                pltpu.async_copy(ones_v, cntacc.at[ia_d], sc0, add=True)
            pltpu.make_async_copy(y.at[ib_s], rows.at[1], sg1).wait()
            pltpu.async_copy(rows.at[1], acc.at[ib_d], ss1, add=True)
            if with_cnt:
                pltpu.async_copy(ones_v, cntacc.at[ib_d], sc1, add=True)
            nbs = lax.rem((j2 + 1) // 2, 2)
            nca = 2 * lax.rem(j2 + 1, 2)
            pltpu.make_async_copy(rows.at[0], acc.at[ia_d], ss0).wait()
            if with_cnt:
                pltpu.make_async_copy(ones_v, cntacc.at[ia_d], sc0).wait()
            @pl.when(j2 + 1 < PAIRS)
            def _():
                pltpu.async_copy(y.at[idxb.at[nbs, nca, 0]], rows.at[0], sg0)
            pltpu.make_async_copy(rows.at[1], acc.at[ib_d], ss1).wait()
            if with_cnt:
                pltpu.make_async_copy(ones_v, cntacc.at[ib_d], sc1).wait()
            @pl.when(j2 + 1 < PAIRS)
            def _():
                pltpu.async_copy(y.at[idxb.at[nbs, nca + 1, 0]], rows.at[1],
                                 sg1)
            return carry

        lax.fori_loop(0, PAIRS, pair, 0)
        plsc.subcore_barrier()

        for t in range(INIT_STEPS):
            r0 = off + t * CHUNK
            pltpu.sync_copy(acc.at[pl.ds(r0, CHUNK)], rows.at[0])
            pltpu.sync_copy(rows.at[0], out.at[c, pl.ds(r0, CHUNK)])
            if with_cnt:
                pltpu.sync_copy(cntacc.at[pl.ds(r0, CHUNK)], ones_v)
                pltpu.sync_copy(ones_v, cntout.at[c, pl.ds(r0, CHUNK)])

    if with_cnt:
        def body_cnt(y, eR, z, z16, ones16, out, cntout, acc, idxb, rows,
                     sg0, sg1, ss0, ss1, cntacc, ones_v, sc0, sc1):
            body(y, eR, z, z16, ones16, out, cntout, acc, idxb, rows,
                 sg0, sg1, ss0, ss1, cntacc, ones_v, sc0, sc1)
        fn = pl.kernel(body_cnt, mesh=mesh, out_type=out_type,
                       scratch_types=scratch,
                       compiler_params=pltpu.CompilerParams(
                           use_tc_tiling_on_sc=False))
        return fn
    else:
        def body_nocnt(y, eR, z, out, acc, idxb, rows, sg0, sg1, ss0, ss1):
            body(y, eR, z, None, None, out, None, acc, idxb, rows,
                 sg0, sg1, ss0, ss1)
        fn = pl.kernel(body_nocnt, mesh=mesh, out_type=out_type,
                       scratch_types=scratch,
                       compiler_params=pltpu.CompilerParams(
                           use_tc_tiling_on_sc=False))
        return fn


def _tc_pre_body(x_ref, wl_ref, wr_ref, b_ref, y_ref, r_ref):
    xb = x_ref[...]
    y_ref[...] = jnp.dot(xb, wl_ref[...], preferred_element_type=jnp.float32)
    r_ref[...] = (
        jnp.dot(xb, wr_ref[...], preferred_element_type=jnp.float32)
        + b_ref[...]
    )


def _tc_mid_body(p0_ref, p1_ref, c0_ref, c1_ref, r1_ref, w2l_ref, w2r_ref,
                 b2_ref, y2_ref, r2_ref):
    cnt = jnp.maximum(c0_ref[:, 0:1] + c1_ref[:, 0:1], 1.0)
    h = jnp.maximum((p0_ref[...] + p1_ref[...]) / cnt + r1_ref[...], 0.0)
    y2_ref[...] = jnp.dot(h, w2l_ref[...], preferred_element_type=jnp.float32)
    r2_ref[...] = (
        jnp.dot(h, w2r_ref[...], preferred_element_type=jnp.float32)
        + b2_ref[...]
    )


def _tc_post_body(q0_ref, q1_ref, c0_ref, c1_ref, r2_ref, out_ref):
    cnt = jnp.maximum(c0_ref[:, 0:1] + c1_ref[:, 0:1], 1.0)
    out_ref[...] = (q0_ref[...] + q1_ref[...]) / cnt + r2_ref[...]


def kernel(x, edge_index, W1_l, b1, W1_r, W2_l, b2, W2_r):
    src = edge_index[0].astype(jnp.int32)
    dst = edge_index[1].astype(jnp.int32)
    pad = E_PAD - N_EDGES
    srcR = jnp.concatenate([src, jnp.zeros((pad,), jnp.int32)]).reshape(
        NW, CHUNKS, 1, CHUNK)
    dstR = jnp.concatenate(
        [dst, jnp.full((pad,), N_NODES, jnp.int32)]).reshape(
        NW, CHUNKS, 1, CHUNK)
    eR = jnp.concatenate([srcR, dstR], axis=2).reshape(
        NW, NBLK, BLK, 2, CHUNK)
    z128 = jnp.zeros((CHUNK, HIDDEN), jnp.float32)
    z64 = jnp.zeros((CHUNK, NUM_CLASSES), jnp.float32)
    z16 = jnp.zeros((CHUNK, 16), jnp.float32)
    ones16 = jnp.ones((CHUNK, 16), jnp.float32)

    y1, r1 = pl.pallas_call(
        _tc_pre_body,
        out_shape=[
            jax.ShapeDtypeStruct((N_NODES, HIDDEN), jnp.float32),
            jax.ShapeDtypeStruct((N_NODES, HIDDEN), jnp.float32),
        ],
    )(x, W1_l, W1_r, b1.reshape(1, HIDDEN))

    p, cntp = _sc_aggregate(HIDDEN, True)(y1, eR, z128, z16, ones16)
    p0 = p[0, :N_NODES]
    p1 = p[1, :N_NODES]
    c0 = cntp[0, :N_NODES]
    c1 = cntp[1, :N_NODES]

    y2, r2 = pl.pallas_call(
        _tc_mid_body,
        out_shape=[
            jax.ShapeDtypeStruct((N_NODES, NUM_CLASSES), jnp.float32),
            jax.ShapeDtypeStruct((N_NODES, NUM_CLASSES), jnp.float32),
        ],
    )(p0, p1, c0, c1, r1, W2_l, W2_r, b2.reshape(1, NUM_CLASSES))

    (q,) = _sc_aggregate(NUM_CLASSES, False)(y2, eR, z64)
    q0 = q[0, :N_NODES]
    q1 = q[1, :N_NODES]

    out = pl.pallas_call(
        _tc_post_body,
        out_shape=jax.ShapeDtypeStruct((N_NODES, NUM_CLASSES), jnp.float32),
    )(q0, q1, c0, c1, r2)
    return out


# speedup vs baseline: 2.0971x; 2.0971x over previous
import jax
import jax.numpy as jnp
from jax import lax
from jax.experimental import pallas as pl
from jax.experimental.pallas import tpu as pltpu
from jax.experimental.pallas import tpu_sc as plsc

N_NODES = 10000
N_EDGES = 320000
IN_FEATS = 128
HIDDEN = 128
NUM_CLASSES = 64
HALF = HIDDEN // 2

NC = 2
NS = 16
NW = NC * NS
CHUNK = 128
BLK = 4
NBLK = 20
CHUNKS = BLK * NBLK
E_PAD = NW * CHUNKS * CHUNK
ROWS_PER_TILE = 640
N_PAD = NS * ROWS_PER_TILE
INIT_STEPS = ROWS_PER_TILE // CHUNK
TCB = 1024
TCG = N_PAD // TCB


def _sc_aggregate(d, with_cnt, by_s):
    mesh = plsc.VectorSubcoreMesh(core_axis_name="c", subcore_axis_name="s")
    n_blk = 2 * NBLK if by_s else NBLK
    pairs = n_blk * BLK // 2
    out_type = [jax.ShapeDtypeStruct((NC, N_PAD, d), jnp.float32)]
    scratch = [
        pltpu.VMEM_SHARED((N_PAD, d), jnp.float32),
        pltpu.VMEM_SHARED((N_PAD, d), jnp.float32),
        pltpu.VMEM((2, BLK, 2, CHUNK), jnp.int32),
        pltpu.VMEM((2, CHUNK, d), jnp.float32),
        pltpu.SemaphoreType.DMA,
        pltpu.SemaphoreType.DMA,
        pltpu.SemaphoreType.DMA,
        pltpu.SemaphoreType.DMA,
    ]
    if with_cnt:
        out_type.append(jax.ShapeDtypeStruct((NC, N_PAD, 16), jnp.float32))
        scratch += [
            pltpu.VMEM_SHARED((N_PAD, 16), jnp.float32),
            pltpu.VMEM((CHUNK, 16), jnp.float32),
            pltpu.SemaphoreType.DMA,
            pltpu.SemaphoreType.DMA,
        ]

    def body(y, eR, z, z16, ones16, out, cntout, acc, ysp, idxb, rows,
             sg0, sg1, ss0, ss1, cntacc=None, ones_v=None, sc0=None,
             sc1=None):
        c = lax.axis_index("c")
        s = lax.axis_index("s")
        w = c * NS + s
        off = s * ROWS_PER_TILE

        pltpu.sync_copy(z, rows.at[0])
        for t in range(INIT_STEPS):
            pltpu.sync_copy(rows.at[0], acc.at[pl.ds(off + t * CHUNK, CHUNK)])
        if with_cnt:
            pltpu.sync_copy(z16, ones_v)
            for t in range(INIT_STEPS):
                pltpu.sync_copy(ones_v, cntacc.at[pl.ds(off + t * CHUNK, CHUNK)])
            pltpu.sync_copy(ones16, ones_v)
        if by_s:
            pltpu.sync_copy(y.at[c, pl.ds(off, ROWS_PER_TILE)],
                            ysp.at[pl.ds(off, ROWS_PER_TILE)])
        else:
            pltpu.sync_copy(y.at[pl.ds(off, ROWS_PER_TILE)],
                            ysp.at[pl.ds(off, ROWS_PER_TILE)])
        plsc.subcore_barrier()

        widx = s if by_s else w

        pltpu.sync_copy(eR.at[widx, 0], idxb.at[0])
        pltpu.async_copy(ysp.at[idxb.at[0, 0, 0]], rows.at[0], sg0)
        pltpu.async_copy(ysp.at[idxb.at[0, 1, 0]], rows.at[1], sg1)

        def pair(j2, carry):
            blk = j2 // 2
            @pl.when(jnp.logical_and(lax.rem(j2, 2) == 0, blk + 1 < n_blk))
            def _():
                pltpu.sync_copy(eR.at[widx, blk + 1],
                                idxb.at[lax.rem(blk + 1, 2)])
            bs = lax.rem(blk, 2)
            ca = 2 * lax.rem(j2, 2)
            ia_s = idxb.at[bs, ca, 0]
            ia_d = idxb.at[bs, ca, 1]
            ib_s = idxb.at[bs, ca + 1, 0]
            ib_d = idxb.at[bs, ca + 1, 1]
            pltpu.make_async_copy(ysp.at[ia_s], rows.at[0], sg0).wait()
            pltpu.async_copy(rows.at[0], acc.at[ia_d], ss0, add=True)
            if with_cnt:
                @pl.when(c == 0)
                def _():
                    pltpu.async_copy(ones_v, cntacc.at[ia_d], sc0, add=True)
            pltpu.make_async_copy(ysp.at[ib_s], rows.at[1], sg1).wait()
            pltpu.async_copy(rows.at[1], acc.at[ib_d], ss1, add=True)
            if with_cnt:
                @pl.when(c == 1)
                def _():
                    pltpu.async_copy(ones_v, cntacc.at[ib_d], sc1, add=True)
            nbs = lax.rem((j2 + 1) // 2, 2)
            nca = 2 * lax.rem(j2 + 1, 2)
            pltpu.make_async_copy(rows.at[0], acc.at[ia_d], ss0).wait()
            if with_cnt:
                @pl.when(c == 0)
                def _():
                    pltpu.make_async_copy(ones_v, cntacc.at[ia_d], sc0).wait()
            @pl.when(j2 + 1 < pairs)
            def _():
                pltpu.async_copy(ysp.at[idxb.at[nbs, nca, 0]], rows.at[0],
                                 sg0)
            pltpu.make_async_copy(rows.at[1], acc.at[ib_d], ss1).wait()
            if with_cnt:
                @pl.when(c == 1)
                def _():
                    pltpu.make_async_copy(ones_v, cntacc.at[ib_d], sc1).wait()
            @pl.when(j2 + 1 < pairs)
            def _():
                pltpu.async_copy(ysp.at[idxb.at[nbs, nca + 1, 0]],
                                 rows.at[1], sg1)
            return carry

        lax.fori_loop(0, pairs, pair, 0)
        plsc.subcore_barrier()

        for t in range(INIT_STEPS):
            r0 = off + t * CHUNK
            pltpu.sync_copy(acc.at[pl.ds(r0, CHUNK)], rows.at[0])
            pltpu.sync_copy(rows.at[0], out.at[c, pl.ds(r0, CHUNK)])
            if with_cnt:
                pltpu.sync_copy(cntacc.at[pl.ds(r0, CHUNK)], ones_v)
                pltpu.sync_copy(ones_v, cntout.at[c, pl.ds(r0, CHUNK)])

    params = pltpu.CompilerParams(use_tc_tiling_on_sc=False)
    if with_cnt:
        def body_cnt(y, eR, z, z16, ones16, out, cntout, acc, ysp, idxb,
                     rows, sg0, sg1, ss0, ss1, cntacc, ones_v, sc0, sc1):
            body(y, eR, z, z16, ones16, out, cntout, acc, ysp, idxb, rows,
                 sg0, sg1, ss0, ss1, cntacc, ones_v, sc0, sc1)
        return pl.kernel(body_cnt, mesh=mesh, out_type=out_type,
                         scratch_types=scratch, compiler_params=params)
    else:
        def body_nocnt(y, eR, z, out, acc, ysp, idxb, rows, sg0, sg1, ss0,
                       ss1):
            body(y, eR, z, None, None, out, None, acc, ysp, idxb, rows,
                 sg0, sg1, ss0, ss1)
        return pl.kernel(body_nocnt, mesh=mesh, out_type=out_type,
                         scratch_types=scratch, compiler_params=params)


def _tc_pre_body(x_ref, wl_ref, wr_ref, b_ref, ys_ref, r_ref):
    xb = x_ref[...]
    y = jnp.dot(xb, wl_ref[...], preferred_element_type=jnp.float32)
    ys_ref[0] = y[:, :HALF]
    ys_ref[1] = y[:, HALF:]
    r_ref[...] = (
        jnp.dot(xb, wr_ref[...], preferred_element_type=jnp.float32)
        + b_ref[...]
    )


def _tc_mid_body(p0_ref, p1_ref, c0_ref, c1_ref, r1_ref, w2l_ref, w2r_ref,
                 b2_ref, y2_ref, r2_ref):
    cnt = jnp.maximum(c0_ref[:, 0:1] + c1_ref[:, 0:1], 1.0)
    mean = jnp.concatenate([p0_ref[...], p1_ref[...]], axis=1) / cnt
    h = jnp.maximum(mean + r1_ref[...], 0.0)
    y2_ref[...] = jnp.dot(h, w2l_ref[...], preferred_element_type=jnp.float32)
    r2_ref[...] = (
        jnp.dot(h, w2r_ref[...], preferred_element_type=jnp.float32)
        + b2_ref[...]
    )


def _tc_post_body(q0_ref, q1_ref, c0_ref, c1_ref, r2_ref, out_ref):
    cnt = jnp.maximum(c0_ref[:, 0:1] + c1_ref[:, 0:1], 1.0)
    out_ref[...] = (q0_ref[...] + q1_ref[...]) / cnt + r2_ref[...]


def kernel(x, edge_index, W1_l, b1, W1_r, W2_l, b2, W2_r):
    src = edge_index[0].astype(jnp.int32)
    dst = edge_index[1].astype(jnp.int32)
    pad = E_PAD - N_EDGES
    srcR = jnp.concatenate([src, jnp.zeros((pad,), jnp.int32)]).reshape(
        NW, CHUNKS, 1, CHUNK)
    dstR = jnp.concatenate(
        [dst, jnp.full((pad,), N_NODES, jnp.int32)]).reshape(
        NW, CHUNKS, 1, CHUNK)
    eAll = jnp.concatenate([srcR, dstR], axis=2)
    eR = eAll.reshape(NW, NBLK, BLK, 2, CHUNK)
    eR2 = eAll.reshape(NS, 2 * NBLK, BLK, 2, CHUNK)
    z64 = jnp.zeros((CHUNK, HALF), jnp.float32)
    z16 = jnp.zeros((CHUNK, 16), jnp.float32)
    ones16 = jnp.ones((CHUNK, 16), jnp.float32)

    y1s, r1 = pl.pallas_call(
        _tc_pre_body,
        grid=(TCG,),
        in_specs=[
            pl.BlockSpec((TCB, IN_FEATS), lambda i: (i, 0)),
            pl.BlockSpec((IN_FEATS, HIDDEN), lambda i: (0, 0)),
            pl.BlockSpec((IN_FEATS, HIDDEN), lambda i: (0, 0)),
            pl.BlockSpec((1, HIDDEN), lambda i: (0, 0)),
        ],
        out_specs=[
            pl.BlockSpec((NC, TCB, HALF), lambda i: (0, i, 0)),
            pl.BlockSpec((TCB, HIDDEN), lambda i: (i, 0)),
        ],
        out_shape=[
            jax.ShapeDtypeStruct((NC, N_PAD, HALF), jnp.float32),
            jax.ShapeDtypeStruct((N_NODES, HIDDEN), jnp.float32),
        ],
    )(x, W1_l, W1_r, b1.reshape(1, HIDDEN))

    p, cntp = _sc_aggregate(HALF, True, True)(y1s, eR2, z64, z16, ones16)
    p0 = p[0, :N_NODES]
    p1 = p[1, :N_NODES]
    c0 = cntp[0, :N_NODES]
    c1 = cntp[1, :N_NODES]

    y2, r2 = pl.pallas_call(
        _tc_mid_body,
        grid=(TCG,),
        in_specs=[
            pl.BlockSpec((TCB, HALF), lambda i: (i, 0)),
            pl.BlockSpec((TCB, HALF), lambda i: (i, 0)),
            pl.BlockSpec((TCB, 16), lambda i: (i, 0)),
            pl.BlockSpec((TCB, 16), lambda i: (i, 0)),
            pl.BlockSpec((TCB, HIDDEN), lambda i: (i, 0)),
            pl.BlockSpec((HIDDEN, NUM_CLASSES), lambda i: (0, 0)),
            pl.BlockSpec((HIDDEN, NUM_CLASSES), lambda i: (0, 0)),
            pl.BlockSpec((1, NUM_CLASSES), lambda i: (0, 0)),
        ],
        out_specs=[
            pl.BlockSpec((TCB, NUM_CLASSES), lambda i: (i, 0)),
            pl.BlockSpec((TCB, NUM_CLASSES), lambda i: (i, 0)),
        ],
        out_shape=[
            jax.ShapeDtypeStruct((N_PAD, NUM_CLASSES), jnp.float32),
            jax.ShapeDtypeStruct((N_NODES, NUM_CLASSES), jnp.float32),
        ],
    )(p0, p1, c0, c1, r1, W2_l, W2_r, b2.reshape(1, NUM_CLASSES))

    (q,) = _sc_aggregate(NUM_CLASSES, False, False)(y2, eR, z64)
    q0 = q[0, :N_NODES]
    q1 = q[1, :N_NODES]

    out = pl.pallas_call(
        _tc_post_body,
        grid=(TCG,),
        in_specs=[
            pl.BlockSpec((TCB, NUM_CLASSES), lambda i: (i, 0)),
            pl.BlockSpec((TCB, NUM_CLASSES), lambda i: (i, 0)),
            pl.BlockSpec((TCB, 16), lambda i: (i, 0)),
            pl.BlockSpec((TCB, 16), lambda i: (i, 0)),
            pl.BlockSpec((TCB, NUM_CLASSES), lambda i: (i, 0)),
        ],
        out_specs=pl.BlockSpec((TCB, NUM_CLASSES), lambda i: (i, 0)),
        out_shape=jax.ShapeDtypeStruct((N_NODES, NUM_CLASSES), jnp.float32),
    )(q0, q1, c0, c1, r2)
    return out


# speedup vs baseline: 2.1831x; 1.0410x over previous
import jax
import jax.numpy as jnp
from jax import lax
from jax.experimental import pallas as pl
from jax.experimental.pallas import tpu as pltpu
from jax.experimental.pallas import tpu_sc as plsc

N_NODES = 10000
N_EDGES = 320000
IN_FEATS = 128
HIDDEN = 128
NUM_CLASSES = 64
HALF = HIDDEN // 2

NC = 2
NS = 16
NW = NC * NS
CHUNK = 128
BLK = 4
NBLK = 20
CHUNKS = BLK * NBLK
E_PAD = NW * CHUNKS * CHUNK
ROWS_PER_TILE = 640
N_PAD = NS * ROWS_PER_TILE
INIT_STEPS = ROWS_PER_TILE // CHUNK
TCB = 1024
TCG = N_PAD // TCB


def _sc_aggregate(d, with_cnt, by_s):
    mesh = plsc.VectorSubcoreMesh(core_axis_name="c", subcore_axis_name="s")
    n_blk = 2 * NBLK if by_s else NBLK
    pairs = n_blk * BLK // 2
    out_type = [jax.ShapeDtypeStruct((NC, N_PAD, d), jnp.float32)]
    scratch = [
        pltpu.VMEM_SHARED((N_PAD, d), jnp.float32),
        pltpu.VMEM_SHARED((N_PAD, d), jnp.float32),
        pltpu.VMEM((2, BLK, 2, CHUNK), jnp.int32),
        pltpu.VMEM((2, CHUNK, d), jnp.float32),
        pltpu.SemaphoreType.DMA,
        pltpu.SemaphoreType.DMA,
        pltpu.SemaphoreType.DMA,
        pltpu.SemaphoreType.DMA,
    ]
    if with_cnt:
        out_type.append(jax.ShapeDtypeStruct((NC, N_PAD, 16), jnp.float32))
        scratch += [
            pltpu.VMEM_SHARED((N_PAD, 16), jnp.float32),
            pltpu.VMEM((CHUNK, 16), jnp.float32),
            pltpu.SemaphoreType.DMA,
            pltpu.SemaphoreType.DMA,
        ]

    def body(y, eR, z, z16, ones16, out, cntout, acc, ysp, idxb, rows,
             sg0, sg1, ss0, ss1, cntacc=None, ones_v=None, sc0=None,
             sc1=None):
        c = lax.axis_index("c")
        s = lax.axis_index("s")
        w = c * NS + s
        off = s * ROWS_PER_TILE

        pltpu.sync_copy(z, rows.at[0])
        for t in range(INIT_STEPS):
            pltpu.sync_copy(rows.at[0], acc.at[pl.ds(off + t * CHUNK, CHUNK)])
        if with_cnt:
            pltpu.sync_copy(z16, ones_v)
            for t in range(INIT_STEPS):
                pltpu.sync_copy(ones_v, cntacc.at[pl.ds(off + t * CHUNK, CHUNK)])
            pltpu.sync_copy(ones16, ones_v)
        if by_s:
            pltpu.sync_copy(y.at[c, pl.ds(off, ROWS_PER_TILE)],
                            ysp.at[pl.ds(off, ROWS_PER_TILE)])
        else:
            pltpu.sync_copy(y.at[pl.ds(off, ROWS_PER_TILE)],
                            ysp.at[pl.ds(off, ROWS_PER_TILE)])
        plsc.subcore_barrier()

        widx = s if by_s else w

        pltpu.sync_copy(eR.at[widx, 0], idxb.at[0])
        pltpu.async_copy(ysp.at[idxb.at[0, 0, 0]], rows.at[0], sg0)
        pltpu.async_copy(ysp.at[idxb.at[0, 1, 0]], rows.at[1], sg1)

        def pair(j2, carry):
            blk = j2 // 2
            @pl.when(jnp.logical_and(lax.rem(j2, 2) == 0, blk + 1 < n_blk))
            def _():
                pltpu.sync_copy(eR.at[widx, blk + 1],
                                idxb.at[lax.rem(blk + 1, 2)])
            bs = lax.rem(blk, 2)
            ca = 2 * lax.rem(j2, 2)
            ia_s = idxb.at[bs, ca, 0]
            ia_d = idxb.at[bs, ca, 1]
            ib_s = idxb.at[bs, ca + 1, 0]
            ib_d = idxb.at[bs, ca + 1, 1]
            pltpu.make_async_copy(ysp.at[ia_s], rows.at[0], sg0).wait()
            pltpu.async_copy(rows.at[0], acc.at[ia_d], ss0, add=True)
            if with_cnt:
                @pl.when(c == 0)
                def _():
                    pltpu.async_copy(ones_v, cntacc.at[ia_d], sc0, add=True)
            pltpu.make_async_copy(ysp.at[ib_s], rows.at[1], sg1).wait()
            pltpu.async_copy(rows.at[1], acc.at[ib_d], ss1, add=True)
            if with_cnt:
                @pl.when(c == 1)
                def _():
                    pltpu.async_copy(ones_v, cntacc.at[ib_d], sc1, add=True)
            nbs = lax.rem((j2 + 1) // 2, 2)
            nca = 2 * lax.rem(j2 + 1, 2)
            pltpu.make_async_copy(rows.at[0], acc.at[ia_d], ss0).wait()
            if with_cnt:
                @pl.when(c == 0)
                def _():
                    pltpu.make_async_copy(ones_v, cntacc.at[ia_d], sc0).wait()
            @pl.when(j2 + 1 < pairs)
            def _():
                pltpu.async_copy(ysp.at[idxb.at[nbs, nca, 0]], rows.at[0],
                                 sg0)
            pltpu.make_async_copy(rows.at[1], acc.at[ib_d], ss1).wait()
            if with_cnt:
                @pl.when(c == 1)
                def _():
                    pltpu.make_async_copy(ones_v, cntacc.at[ib_d], sc1).wait()
            @pl.when(j2 + 1 < pairs)
            def _():
                pltpu.async_copy(ysp.at[idxb.at[nbs, nca + 1, 0]],
                                 rows.at[1], sg1)
            return carry

        lax.fori_loop(0, pairs, pair, 0)
        plsc.subcore_barrier()

        for t in range(INIT_STEPS):
            r0 = off + t * CHUNK
            pltpu.sync_copy(acc.at[pl.ds(r0, CHUNK)], rows.at[0])
            pltpu.sync_copy(rows.at[0], out.at[c, pl.ds(r0, CHUNK)])
            if with_cnt:
                pltpu.sync_copy(cntacc.at[pl.ds(r0, CHUNK)], ones_v)
                pltpu.sync_copy(ones_v, cntout.at[c, pl.ds(r0, CHUNK)])

    params = pltpu.CompilerParams(use_tc_tiling_on_sc=False)
    if with_cnt:
        def body_cnt(y, eR, z, z16, ones16, out, cntout, acc, ysp, idxb,
                     rows, sg0, sg1, ss0, ss1, cntacc, ones_v, sc0, sc1):
            body(y, eR, z, z16, ones16, out, cntout, acc, ysp, idxb, rows,
                 sg0, sg1, ss0, ss1, cntacc, ones_v, sc0, sc1)
        return pl.kernel(body_cnt, mesh=mesh, out_type=out_type,
                         scratch_types=scratch, compiler_params=params)
    else:
        def body_nocnt(y, eR, z, out, acc, ysp, idxb, rows, sg0, sg1, ss0,
                       ss1):
            body(y, eR, z, None, None, out, None, acc, ysp, idxb, rows,
                 sg0, sg1, ss0, ss1)
        return pl.kernel(body_nocnt, mesh=mesh, out_type=out_type,
                         scratch_types=scratch, compiler_params=params)


def _tc_pre_body(x_ref, wl_ref, wr_ref, b_ref, ys_ref, r_ref):
    xb = x_ref[...]
    y = jnp.dot(xb, wl_ref[...], preferred_element_type=jnp.float32)
    ys_ref[0] = y[:, :HALF]
    ys_ref[1] = y[:, HALF:]
    r_ref[...] = (
        jnp.dot(xb, wr_ref[...], preferred_element_type=jnp.float32)
        + b_ref[...]
    )


def _tc_mid_body(p0_ref, p1_ref, c0_ref, c1_ref, r1_ref, w2l_ref, w2r_ref,
                 b2_ref, y2_ref, r2_ref):
    cnt = jnp.maximum(c0_ref[0, :, 0:1] + c1_ref[0, :, 0:1], 1.0)
    mean = jnp.concatenate([p0_ref[0], p1_ref[0]], axis=1) / cnt
    h = jnp.maximum(mean + r1_ref[...], 0.0)
    y2_ref[...] = jnp.dot(h, w2l_ref[...], preferred_element_type=jnp.float32)
    r2_ref[...] = (
        jnp.dot(h, w2r_ref[...], preferred_element_type=jnp.float32)
        + b2_ref[...]
    )


def _tc_post_body(q0_ref, q1_ref, c0_ref, c1_ref, r2_ref, out_ref):
    cnt = jnp.maximum(c0_ref[0, :, 0:1] + c1_ref[0, :, 0:1], 1.0)
    out_ref[...] = (q0_ref[0] + q1_ref[0]) / cnt + r2_ref[...]


def kernel(x, edge_index, W1_l, b1, W1_r, W2_l, b2, W2_r):
    src = edge_index[0].astype(jnp.int32)
    dst = edge_index[1].astype(jnp.int32)
    pad = E_PAD - N_EDGES
    srcR = jnp.concatenate([src, jnp.zeros((pad,), jnp.int32)]).reshape(
        NW, CHUNKS, 1, CHUNK)
    dstR = jnp.concatenate(
        [dst, jnp.full((pad,), N_NODES, jnp.int32)]).reshape(
        NW, CHUNKS, 1, CHUNK)
    eAll = jnp.concatenate([srcR, dstR], axis=2)
    eR = eAll.reshape(NW, NBLK, BLK, 2, CHUNK)
    eR2 = eAll.reshape(NS, 2 * NBLK, BLK, 2, CHUNK)
    z64 = jnp.zeros((CHUNK, HALF), jnp.float32)
    z16 = jnp.zeros((CHUNK, 16), jnp.float32)
    ones16 = jnp.ones((CHUNK, 16), jnp.float32)

    y1s, r1 = pl.pallas_call(
        _tc_pre_body,
        grid=(TCG,),
        in_specs=[
            pl.BlockSpec((TCB, IN_FEATS), lambda i: (i, 0)),
            pl.BlockSpec((IN_FEATS, HIDDEN), lambda i: (0, 0)),
            pl.BlockSpec((IN_FEATS, HIDDEN), lambda i: (0, 0)),
            pl.BlockSpec((1, HIDDEN), lambda i: (0, 0)),
        ],
        out_specs=[
            pl.BlockSpec((NC, TCB, HALF), lambda i: (0, i, 0)),
            pl.BlockSpec((TCB, HIDDEN), lambda i: (i, 0)),
        ],
        out_shape=[
            jax.ShapeDtypeStruct((NC, N_PAD, HALF), jnp.float32),
            jax.ShapeDtypeStruct((N_NODES, HIDDEN), jnp.float32),
        ],
    )(x, W1_l, W1_r, b1.reshape(1, HIDDEN))

    p, cntp = _sc_aggregate(HALF, True, True)(y1s, eR2, z64, z16, ones16)

    y2, r2 = pl.pallas_call(
        _tc_mid_body,
        grid=(TCG,),
        in_specs=[
            pl.BlockSpec((1, TCB, HALF), lambda i: (0, i, 0)),
            pl.BlockSpec((1, TCB, HALF), lambda i: (1, i, 0)),
            pl.BlockSpec((1, TCB, 16), lambda i: (0, i, 0)),
            pl.BlockSpec((1, TCB, 16), lambda i: (1, i, 0)),
            pl.BlockSpec((TCB, HIDDEN), lambda i: (i, 0)),
            pl.BlockSpec((HIDDEN, NUM_CLASSES), lambda i: (0, 0)),
            pl.BlockSpec((HIDDEN, NUM_CLASSES), lambda i: (0, 0)),
            pl.BlockSpec((1, NUM_CLASSES), lambda i: (0, 0)),
        ],
        out_specs=[
            pl.BlockSpec((TCB, NUM_CLASSES), lambda i: (i, 0)),
            pl.BlockSpec((TCB, NUM_CLASSES), lambda i: (i, 0)),
        ],
        out_shape=[
            jax.ShapeDtypeStruct((N_PAD, NUM_CLASSES), jnp.float32),
            jax.ShapeDtypeStruct((N_NODES, NUM_CLASSES), jnp.float32),
        ],
    )(p, p, cntp, cntp, r1, W2_l, W2_r, b2.reshape(1, NUM_CLASSES))

    (q,) = _sc_aggregate(NUM_CLASSES, False, False)(y2, eR, z64)

    out = pl.pallas_call(
        _tc_post_body,
        grid=(TCG,),
        in_specs=[
            pl.BlockSpec((1, TCB, NUM_CLASSES), lambda i: (0, i, 0)),
            pl.BlockSpec((1, TCB, NUM_CLASSES), lambda i: (1, i, 0)),
            pl.BlockSpec((1, TCB, 16), lambda i: (0, i, 0)),
            pl.BlockSpec((1, TCB, 16), lambda i: (1, i, 0)),
            pl.BlockSpec((TCB, NUM_CLASSES), lambda i: (i, 0)),
        ],
        out_specs=pl.BlockSpec((TCB, NUM_CLASSES), lambda i: (i, 0)),
        out_shape=jax.ShapeDtypeStruct((N_NODES, NUM_CLASSES), jnp.float32),
    )(q, q, cntp, cntp, r2)
    return out


# speedup vs baseline: 2.2346x; 1.0236x over previous
import jax
import jax.numpy as jnp
from jax import lax
from jax.experimental import pallas as pl
from jax.experimental.pallas import tpu as pltpu
from jax.experimental.pallas import tpu_sc as plsc

N_NODES = 10000
N_EDGES = 320000
IN_FEATS = 128
HIDDEN = 128
NUM_CLASSES = 64
HALF = HIDDEN // 2

NC = 2
NS = 16
NW = NC * NS
CHUNK = 128
BLK = 4
NBLK = 20
CHUNKS = BLK * NBLK
E_PAD = NW * CHUNKS * CHUNK
ROWS_PER_TILE = 640
N_PAD = NS * ROWS_PER_TILE
INIT_STEPS = ROWS_PER_TILE // CHUNK
TCB = 1024
TCG = N_PAD // TCB


def _sc_aggregate(d, with_cnt, by_s):
    mesh = plsc.VectorSubcoreMesh(core_axis_name="c", subcore_axis_name="s")
    n_blk = 2 * NBLK if by_s else NBLK
    pairs = n_blk * BLK // 2
    out_type = [jax.ShapeDtypeStruct((NC, N_PAD, d), jnp.float32)]
    scratch = [
        pltpu.VMEM_SHARED((N_PAD, d), jnp.float32),
        pltpu.VMEM_SHARED((N_PAD, d), jnp.float32),
        pltpu.VMEM((4, BLK, 2, CHUNK), jnp.int32),
        pltpu.VMEM((2, CHUNK, d), jnp.float32),
        pltpu.SemaphoreType.DMA,
        pltpu.SemaphoreType.DMA,
        pltpu.SemaphoreType.DMA,
        pltpu.SemaphoreType.DMA,
        pltpu.SemaphoreType.DMA,
        pltpu.SemaphoreType.DMA,
        pltpu.SemaphoreType.DMA,
        pltpu.SemaphoreType.DMA,
    ]
    if with_cnt:
        out_type.append(jax.ShapeDtypeStruct((NC, N_PAD, 16), jnp.float32))
        scratch += [
            pltpu.VMEM_SHARED((N_PAD, 16), jnp.float32),
            pltpu.VMEM((CHUNK, 16), jnp.float32),
            pltpu.SemaphoreType.DMA,
            pltpu.SemaphoreType.DMA,
        ]

    def body(y, eR, z, z16, ones16, out, cntout, acc, ysp, idxb, rows,
             sg0, sg1, ss0, ss1, si0, si1, si2, si3, cntacc=None,
             ones_v=None, sc0=None, sc1=None):
        sis = (si0, si1, si2, si3)
        c = lax.axis_index("c")
        s = lax.axis_index("s")
        w = c * NS + s
        off = s * ROWS_PER_TILE

        pltpu.sync_copy(z, rows.at[0])
        for t in range(INIT_STEPS):
            pltpu.sync_copy(rows.at[0], acc.at[pl.ds(off + t * CHUNK, CHUNK)])
        if with_cnt:
            pltpu.sync_copy(z16, ones_v)
            for t in range(INIT_STEPS):
                pltpu.sync_copy(ones_v, cntacc.at[pl.ds(off + t * CHUNK, CHUNK)])
            pltpu.sync_copy(ones16, ones_v)
        if by_s:
            pltpu.sync_copy(y.at[c, pl.ds(off, ROWS_PER_TILE)],
                            ysp.at[pl.ds(off, ROWS_PER_TILE)])
        else:
            pltpu.sync_copy(y.at[pl.ds(off, ROWS_PER_TILE)],
                            ysp.at[pl.ds(off, ROWS_PER_TILE)])
        plsc.subcore_barrier()

        widx = s if by_s else w

        pltpu.sync_copy(eR.at[widx, 0], idxb.at[0])
        pltpu.async_copy(eR.at[widx, 1], idxb.at[1], si1)
        pltpu.async_copy(eR.at[widx, 2], idxb.at[2], si2)
        pltpu.async_copy(ysp.at[idxb.at[0, 0, 0]], rows.at[0], sg0)
        pltpu.async_copy(ysp.at[idxb.at[0, 1, 0]], rows.at[1], sg1)

        def pair(j2, carry):
            blk = j2 // 2
            even = lax.rem(j2, 2) == 0
            for k in range(4):
                @pl.when(jnp.logical_and(
                        jnp.logical_and(even, blk + 1 < n_blk),
                        lax.rem(blk + 1, 4) == k))
                def _(k=k):
                    pltpu.make_async_copy(eR.at[widx, 0], idxb.at[k],
                                          sis[k]).wait()
                @pl.when(jnp.logical_and(
                        jnp.logical_and(even, blk + 3 < n_blk),
                        lax.rem(blk + 3, 4) == k))
                def _(k=k):
                    pltpu.async_copy(eR.at[widx, blk + 3], idxb.at[k],
                                     sis[k])
            bs = lax.rem(blk, 4)
            ca = 2 * lax.rem(j2, 2)
            ia_s = idxb.at[bs, ca, 0]
            ia_d = idxb.at[bs, ca, 1]
            ib_s = idxb.at[bs, ca + 1, 0]
            ib_d = idxb.at[bs, ca + 1, 1]
            pltpu.make_async_copy(ysp.at[ia_s], rows.at[0], sg0).wait()
            pltpu.async_copy(rows.at[0], acc.at[ia_d], ss0, add=True)
            if with_cnt:
                @pl.when(c == 0)
                def _():
                    pltpu.async_copy(ones_v, cntacc.at[ia_d], sc0, add=True)
            pltpu.make_async_copy(ysp.at[ib_s], rows.at[1], sg1).wait()
            pltpu.async_copy(rows.at[1], acc.at[ib_d], ss1, add=True)
            if with_cnt:
                @pl.when(c == 1)
                def _():
                    pltpu.async_copy(ones_v, cntacc.at[ib_d], sc1, add=True)
            nbs = lax.rem((j2 + 1) // 2, 4)
            nca = 2 * lax.rem(j2 + 1, 2)
            pltpu.make_async_copy(rows.at[0], acc.at[ia_d], ss0).wait()
            if with_cnt:
                @pl.when(c == 0)
                def _():
                    pltpu.make_async_copy(ones_v, cntacc.at[ia_d], sc0).wait()
            @pl.when(j2 + 1 < pairs)
            def _():
                pltpu.async_copy(ysp.at[idxb.at[nbs, nca, 0]], rows.at[0],
                                 sg0)
            pltpu.make_async_copy(rows.at[1], acc.at[ib_d], ss1).wait()
            if with_cnt:
                @pl.when(c == 1)
                def _():
                    pltpu.make_async_copy(ones_v, cntacc.at[ib_d], sc1).wait()
            @pl.when(j2 + 1 < pairs)
            def _():
                pltpu.async_copy(ysp.at[idxb.at[nbs, nca + 1, 0]],
                                 rows.at[1], sg1)
            return carry

        lax.fori_loop(0, pairs, pair, 0)
        plsc.subcore_barrier()

        for t in range(INIT_STEPS):
            r0 = off + t * CHUNK
            pltpu.sync_copy(acc.at[pl.ds(r0, CHUNK)], rows.at[0])
            pltpu.sync_copy(rows.at[0], out.at[c, pl.ds(r0, CHUNK)])
            if with_cnt:
                pltpu.sync_copy(cntacc.at[pl.ds(r0, CHUNK)], ones_v)
                pltpu.sync_copy(ones_v, cntout.at[c, pl.ds(r0, CHUNK)])

    params = pltpu.CompilerParams(use_tc_tiling_on_sc=False)
    if with_cnt:
        def body_cnt(y, eR, z, z16, ones16, out, cntout, acc, ysp, idxb,
                     rows, sg0, sg1, ss0, ss1, si0, si1, si2, si3, cntacc,
                     ones_v, sc0, sc1):
            body(y, eR, z, z16, ones16, out, cntout, acc, ysp, idxb, rows,
                 sg0, sg1, ss0, ss1, si0, si1, si2, si3, cntacc, ones_v,
                 sc0, sc1)
        return pl.kernel(body_cnt, mesh=mesh, out_type=out_type,
                         scratch_types=scratch, compiler_params=params)
    else:
        def body_nocnt(y, eR, z, out, acc, ysp, idxb, rows, sg0, sg1, ss0,
                       ss1, si0, si1, si2, si3):
            body(y, eR, z, None, None, out, None, acc, ysp, idxb, rows,
                 sg0, sg1, ss0, ss1, si0, si1, si2, si3)
        return pl.kernel(body_nocnt, mesh=mesh, out_type=out_type,
                         scratch_types=scratch, compiler_params=params)


def _tc_pre_body(x_ref, wl_ref, wr_ref, b_ref, ys_ref, r_ref):
    xb = x_ref[...]
    y = jnp.dot(xb, wl_ref[...], preferred_element_type=jnp.float32)
    ys_ref[0] = y[:, :HALF]
    ys_ref[1] = y[:, HALF:]
    r_ref[...] = (
        jnp.dot(xb, wr_ref[...], preferred_element_type=jnp.float32)
        + b_ref[...]
    )


def _tc_mid_body(p0_ref, p1_ref, c0_ref, c1_ref, r1_ref, w2l_ref, w2r_ref,
                 b2_ref, y2_ref, r2_ref):
    cnt = jnp.maximum(c0_ref[0, :, 0:1] + c1_ref[0, :, 0:1], 1.0)
    mean = jnp.concatenate([p0_ref[0], p1_ref[0]], axis=1) / cnt
    h = jnp.maximum(mean + r1_ref[...], 0.0)
    y2_ref[...] = jnp.dot(h, w2l_ref[...], preferred_element_type=jnp.float32)
    r2_ref[...] = (
        jnp.dot(h, w2r_ref[...], preferred_element_type=jnp.float32)
        + b2_ref[...]
    )


def _tc_post_body(q0_ref, q1_ref, c0_ref, c1_ref, r2_ref, out_ref):
    cnt = jnp.maximum(c0_ref[0, :, 0:1] + c1_ref[0, :, 0:1], 1.0)
    out_ref[...] = (q0_ref[0] + q1_ref[0]) / cnt + r2_ref[...]


def kernel(x, edge_index, W1_l, b1, W1_r, W2_l, b2, W2_r):
    src = edge_index[0].astype(jnp.int32)
    dst = edge_index[1].astype(jnp.int32)
    pad = E_PAD - N_EDGES
    srcR = jnp.concatenate([src, jnp.zeros((pad,), jnp.int32)]).reshape(
        NW, CHUNKS, 1, CHUNK)
    dstR = jnp.concatenate(
        [dst, jnp.full((pad,), N_NODES, jnp.int32)]).reshape(
        NW, CHUNKS, 1, CHUNK)
    eAll = jnp.concatenate([srcR, dstR], axis=2)
    eR = eAll.reshape(NW, NBLK, BLK, 2, CHUNK)
    eR2 = eAll.reshape(NS, 2 * NBLK, BLK, 2, CHUNK)
    z64 = jnp.zeros((CHUNK, HALF), jnp.float32)
    z16 = jnp.zeros((CHUNK, 16), jnp.float32)
    ones16 = jnp.ones((CHUNK, 16), jnp.float32)

    y1s, r1 = pl.pallas_call(
        _tc_pre_body,
        grid=(TCG,),
        in_specs=[
            pl.BlockSpec((TCB, IN_FEATS), lambda i: (i, 0)),
            pl.BlockSpec((IN_FEATS, HIDDEN), lambda i: (0, 0)),
            pl.BlockSpec((IN_FEATS, HIDDEN), lambda i: (0, 0)),
            pl.BlockSpec((1, HIDDEN), lambda i: (0, 0)),
        ],
        out_specs=[
            pl.BlockSpec((NC, TCB, HALF), lambda i: (0, i, 0)),
            pl.BlockSpec((TCB, HIDDEN), lambda i: (i, 0)),
        ],
        out_shape=[
            jax.ShapeDtypeStruct((NC, N_PAD, HALF), jnp.float32),
            jax.ShapeDtypeStruct((N_NODES, HIDDEN), jnp.float32),
        ],
    )(x, W1_l, W1_r, b1.reshape(1, HIDDEN))

    p, cntp = _sc_aggregate(HALF, True, True)(y1s, eR2, z64, z16, ones16)

    y2, r2 = pl.pallas_call(
        _tc_mid_body,
        grid=(TCG,),
        in_specs=[
            pl.BlockSpec((1, TCB, HALF), lambda i: (0, i, 0)),
            pl.BlockSpec((1, TCB, HALF), lambda i: (1, i, 0)),
            pl.BlockSpec((1, TCB, 16), lambda i: (0, i, 0)),
            pl.BlockSpec((1, TCB, 16), lambda i: (1, i, 0)),
            pl.BlockSpec((TCB, HIDDEN), lambda i: (i, 0)),
            pl.BlockSpec((HIDDEN, NUM_CLASSES), lambda i: (0, 0)),
            pl.BlockSpec((HIDDEN, NUM_CLASSES), lambda i: (0, 0)),
            pl.BlockSpec((1, NUM_CLASSES), lambda i: (0, 0)),
        ],
        out_specs=[
            pl.BlockSpec((TCB, NUM_CLASSES), lambda i: (i, 0)),
            pl.BlockSpec((TCB, NUM_CLASSES), lambda i: (i, 0)),
        ],
        out_shape=[
            jax.ShapeDtypeStruct((N_PAD, NUM_CLASSES), jnp.float32),
            jax.ShapeDtypeStruct((N_NODES, NUM_CLASSES), jnp.float32),
        ],
    )(p, p, cntp, cntp, r1, W2_l, W2_r, b2.reshape(1, NUM_CLASSES))

    (q,) = _sc_aggregate(NUM_CLASSES, False, False)(y2, eR, z64)

    out = pl.pallas_call(
        _tc_post_body,
        grid=(TCG,),
        in_specs=[
            pl.BlockSpec((1, TCB, NUM_CLASSES), lambda i: (0, i, 0)),
            pl.BlockSpec((1, TCB, NUM_CLASSES), lambda i: (1, i, 0)),
            pl.BlockSpec((1, TCB, 16), lambda i: (0, i, 0)),
            pl.BlockSpec((1, TCB, 16), lambda i: (1, i, 0)),
            pl.BlockSpec((TCB, NUM_CLASSES), lambda i: (i, 0)),
        ],
        out_specs=pl.BlockSpec((TCB, NUM_CLASSES), lambda i: (i, 0)),
        out_shape=jax.ShapeDtypeStruct((N_NODES, NUM_CLASSES), jnp.float32),
    )(q, q, cntp, cntp, r2)
    return out
